# baseline (speedup 1.0000x reference)
import functools

import jax
import jax.numpy as jnp
from jax import lax
from jax.experimental import pallas as pl
from jax.experimental.pallas import tpu as pltpu
from jax.experimental.pallas import tpu_sc as plsc

NF = 26
V = 100000
D = 50
BATCH = 16384
NUMF = 13

NC = 2
NS = 16
NW = NC * NS
ROWS = BATCH * NF
RPW = ROWS // NW
CHUNK = 128
NCHUNK = RPW // CHUNK

@functools.cache
def _get_sc_gather():
    mesh = plsc.VectorSubcoreMesh(
        core_axis_name="c", subcore_axis_name="s",
        num_cores=NC, num_subcores=NS)

    @functools.partial(
        pl.kernel,
        out_type=jax.ShapeDtypeStruct((ROWS, D), jnp.float32),
        mesh=mesh,
        scratch_types=[
            pltpu.VMEM((NCHUNK, CHUNK), jnp.int32),
            pltpu.VMEM((CHUNK, D), jnp.float32),
            pltpu.SemaphoreType.DMA,
        ],
        compiler_params=pltpu.CompilerParams(use_tc_tiling_on_sc=False),
    )
    def _sc_gather(table_hbm, idx_hbm, out_hbm, idx_v, rows_v, sem):
        wid = lax.axis_index("c") * NS + lax.axis_index("s")
        base = wid * RPW
        pltpu.sync_copy(idx_hbm.at[wid], idx_v)

        def body(j, carry):
            pltpu.async_copy(table_hbm.at[idx_v.at[j]], rows_v, sem).wait()
            pltpu.sync_copy(
                rows_v, out_hbm.at[pl.ds(base + j * CHUNK, CHUNK)])
            return carry

        lax.fori_loop(0, NCHUNK, body, 0)

    return _sc_gather


def _mlp_body(act_ref, xn_ref, w1a_ref, w1b_ref, b1_ref, w2_ref, b2_ref,
              w3_ref, b3_ref, w4_ref, b4_ref, wskip_ref, out_ref):
    x = act_ref[...]
    xn = xn_ref[...]
    h = jnp.dot(x, w1a_ref[...], preferred_element_type=jnp.float32)
    h += jnp.dot(xn, w1b_ref[...], preferred_element_type=jnp.float32)
    h = jnp.maximum(h + b1_ref[...], 0.0)
    h = jnp.maximum(
        jnp.dot(h, w2_ref[...], preferred_element_type=jnp.float32)
        + b2_ref[...], 0.0)
    h = jnp.maximum(
        jnp.dot(h, w3_ref[...], preferred_element_type=jnp.float32)
        + b3_ref[...], 0.0)
    out = jnp.dot(h, w4_ref[...], preferred_element_type=jnp.float32)
    out_ref[...] = out + b4_ref[...] + xn[:, 0:1] * wskip_ref[0, 0]


def _tc_mlp(act, x_num, W1a, W1b, b1, W2, b2, W3, b3, W4, b4, Wskip):
    BB = 2048
    grid = (BATCH // BB,)
    full = lambda shape: pl.BlockSpec(shape, lambda i: (0,) * len(shape))
    return pl.pallas_call(
        _mlp_body,
        grid=grid,
        in_specs=[
            pl.BlockSpec((BB, NF * D), lambda i: (i, 0)),
            pl.BlockSpec((BB, NUMF), lambda i: (i, 0)),
            full(W1a.shape),
            full(W1b.shape),
            full(b1.shape),
            full(W2.shape),
            full(b2.shape),
            full(W3.shape),
            full(b3.shape),
            full(W4.shape),
            full(b4.shape),
            full(Wskip.shape),
        ],
        out_specs=pl.BlockSpec((BB, 1), lambda i: (i, 0)),
        out_shape=jax.ShapeDtypeStruct((BATCH, 1), jnp.float32),
    )(act, x_num, W1a, W1b, b1, W2, b2, W3, b3, W4, b4, Wskip)


@jax.jit
def kernel(x_cat, x_num, tables, W1, b1, W2, b2, W3, b3, W4, b4, Wskip):
    flat_table = tables.reshape(NF * V, D)
    offsets = (jnp.arange(NF, dtype=jnp.int32) * V)[None, :]
    flat_idx = (x_cat + offsets).reshape(NW, NCHUNK, CHUNK)
    act = _get_sc_gather()(flat_table, flat_idx)
    act = act.reshape(BATCH, NF * D)
    out = _tc_mlp(
        act, x_num,
        W1[:NF * D], W1[NF * D:], b1[None, :],
        W2, b2[None, :], W3, b3[None, :], W4, b4[None, :], Wskip)
    return out
